# Initial kernel scaffold; baseline (speedup 1.0000x reference)
import jax
import jax.numpy as jnp
from jax.experimental import pallas as pl


def kernel(input, embeddings):
    raise NotImplementedError("write your pallas kernel here")


# speedup vs baseline: 1.9055x; 1.9055x over previous
import functools
import jax
import jax.numpy as jnp
from jax import lax
from jax.experimental import pallas as pl
from jax.experimental.pallas import tpu as pltpu
from jax.experimental.pallas import tpu_sc as plsc

_NUM_EMB = 4098
_DIM = 1024
_BATCH = 4
_SEQ = 4096

_NC = 2
_NS = 16
_L = 16
_NW = _NC * _NS

_TOKENS = _BATCH * _SEQ
_PER_W = _TOKENS // _NW
_CH = 32
_NCH = _PER_W // _CH


def _body(ids_hbm, table_hbm, out_hbm, ids_v, pos_v, buf0, buf1, sem0, sem1):
    wid = lax.axis_index("s") * _NC + lax.axis_index("c")
    base = wid * _PER_W
    col_base = lax.rem(base, _SEQ)

    pltpu.sync_copy(ids_hbm.at[pl.ds(base, _PER_W)], ids_v)
    lane = lax.iota(jnp.int32, _L)
    for c in range(_NCH):
        for i in range(_CH // _L):
            off = c * _CH + i * _L
            ids = ids_v[pl.ds(off, _L)]
            col = (col_base + off + 1) + lane
            pos_v[c, pl.ds(i * _L, _L)] = jnp.where(ids != 0, col, 0)

    bufs = (buf0, buf1)
    sems = (sem0, sem1)
    copies = [None, None]
    copies[0] = pltpu.async_copy(table_hbm.at[pos_v.at[0]], bufs[0], sems[0])
    for c in range(_NCH):
        if c + 1 < _NCH:
            copies[(c + 1) % 2] = pltpu.async_copy(
                table_hbm.at[pos_v.at[c + 1]], bufs[(c + 1) % 2],
                sems[(c + 1) % 2])
        copies[c % 2].wait()
        pltpu.sync_copy(bufs[c % 2], out_hbm.at[pl.ds(base + c * _CH, _CH)])


@jax.jit
def _lookup(ids_flat, table):
    mesh = plsc.VectorSubcoreMesh(
        core_axis_name="c", subcore_axis_name="s",
        num_cores=_NC, num_subcores=_NS)
    fn = pl.kernel(
        _body,
        out_type=jax.ShapeDtypeStruct((_TOKENS, _DIM), jnp.float32),
        mesh=mesh,
        scratch_types=[
            pltpu.VMEM((_PER_W,), jnp.int32),
            pltpu.VMEM((_NCH, _CH), jnp.int32),
            pltpu.VMEM((_CH, _DIM), jnp.float32),
            pltpu.VMEM((_CH, _DIM), jnp.float32),
            pltpu.SemaphoreType.DMA,
            pltpu.SemaphoreType.DMA,
        ],
    )
    return fn(ids_flat, table)


def kernel(input, embeddings):
    ids_flat = input.astype(jnp.int32).reshape(_TOKENS)
    out = _lookup(ids_flat, embeddings)
    return out.reshape(_BATCH, _SEQ, _DIM)


# speedup vs baseline: 1.9312x; 1.0135x over previous
import functools
import jax
import jax.numpy as jnp
from jax import lax
from jax.experimental import pallas as pl
from jax.experimental.pallas import tpu as pltpu
from jax.experimental.pallas import tpu_sc as plsc

_NUM_EMB = 4098
_DIM = 1024
_BATCH = 4
_SEQ = 4096

_NC = 2
_NS = 16
_L = 16
_NW = _NC * _NS

_TOKENS = _BATCH * _SEQ
_PER_W = _TOKENS // _NW
_CH = 32
_NCH = _PER_W // _CH


def _body(ids_hbm, table_hbm, out_hbm, ids_v, pos_v, buf0, buf1, buf2,
          gsem0, gsem1, gsem2, ssem0, ssem1, ssem2):
    wid = lax.axis_index("s") * _NC + lax.axis_index("c")
    base = wid * _PER_W
    col_base = lax.rem(base, _SEQ)

    pltpu.sync_copy(ids_hbm.at[pl.ds(base, _PER_W)], ids_v)
    lane = lax.iota(jnp.int32, _L)
    for c in range(_NCH):
        for i in range(_CH // _L):
            off = c * _CH + i * _L
            ids = ids_v[pl.ds(off, _L)]
            col = (col_base + off + 1) + lane
            pos_v[c, pl.ds(i * _L, _L)] = jnp.where(ids != 0, col, 0)

    bufs = (buf0, buf1, buf2)
    gsems = (gsem0, gsem1, gsem2)
    ssems = (ssem0, ssem1, ssem2)
    gcp = [None, None, None]
    scp = [None, None, None]
    gcp[0] = pltpu.async_copy(table_hbm.at[pos_v.at[0]], bufs[0], gsems[0])
    gcp[1] = pltpu.async_copy(table_hbm.at[pos_v.at[1]], bufs[1], gsems[1])
    for c in range(_NCH):
        b = c % 3
        gcp[b].wait()
        scp[b] = pltpu.async_copy(
            bufs[b], out_hbm.at[pl.ds(base + c * _CH, _CH)], ssems[b])
        n = c + 2
        if n < _NCH:
            nb = n % 3
            if scp[nb] is not None:
                scp[nb].wait()
                scp[nb] = None
            gcp[nb] = pltpu.async_copy(
                table_hbm.at[pos_v.at[n]], bufs[nb], gsems[nb])
    for b in range(3):
        if scp[b] is not None:
            scp[b].wait()


@jax.jit
def _lookup(ids_flat, table):
    mesh = plsc.VectorSubcoreMesh(
        core_axis_name="c", subcore_axis_name="s",
        num_cores=_NC, num_subcores=_NS)
    fn = pl.kernel(
        _body,
        out_type=jax.ShapeDtypeStruct((_TOKENS, _DIM), jnp.float32),
        mesh=mesh,
        scratch_types=[
            pltpu.VMEM((_PER_W,), jnp.int32),
            pltpu.VMEM((_NCH, _CH), jnp.int32),
            pltpu.VMEM((_CH, _DIM), jnp.float32),
            pltpu.VMEM((_CH, _DIM), jnp.float32),
            pltpu.VMEM((_CH, _DIM), jnp.float32),
            pltpu.SemaphoreType.DMA,
            pltpu.SemaphoreType.DMA,
            pltpu.SemaphoreType.DMA,
            pltpu.SemaphoreType.DMA,
            pltpu.SemaphoreType.DMA,
            pltpu.SemaphoreType.DMA,
        ],
    )
    return fn(ids_flat, table)


def kernel(input, embeddings):
    ids_flat = input.astype(jnp.int32).reshape(_TOKENS)
    out = _lookup(ids_flat, embeddings)
    return out.reshape(_BATCH, _SEQ, _DIM)
